# baseline (speedup 1.0000x reference)
import functools

import jax
import jax.numpy as jnp
from jax import lax
from jax.experimental import pallas as pl
from jax.experimental.pallas import tpu as pltpu
from jax.experimental.pallas import tpu_sc as plsc

B = 16384
L = 50
D = 64
DP = D
LAM = 0.001

NC = 2
NS = 16
NW = NC * NS
U_PER_W = B // NW
CU = 8
ROWS = CU * L
CHUNKS = U_PER_W // CU
GSUB = 4
GLEN = ROWS // GSUB
PROWS_PW = U_PER_W * L // GLEN
GROUPS = (ROWS + 15) // 16
RPAD = GROUPS * 16

_mesh = plsc.VectorSubcoreMesh(core_axis_name="c", subcore_axis_name="s")


@functools.partial(
    pl.kernel,
    mesh=_mesh,
    compiler_params=pltpu.CompilerParams(
        needs_layout_passes=False, use_tc_tiling_on_sc=False),
    out_type=[
        jax.ShapeDtypeStruct((B * L,), jnp.float32),
        jax.ShapeDtypeStruct((NW, 16), jnp.float32),
    ],
    scratch_types=[
        pltpu.VMEM((PROWS_PW, GLEN), jnp.int32),
        pltpu.VMEM((2, CU, DP), jnp.float32),
        pltpu.VMEM((2, ROWS, DP), jnp.float32),
        pltpu.VMEM((RPAD * 16,), jnp.float32),
        pltpu.VMEM((RPAD,), jnp.float32),
        pltpu.VMEM((16,), jnp.float32),
        pltpu.SemaphoreType.DMA,
        pltpu.SemaphoreType.DMA,
        pltpu.SemaphoreType.DMA,
        pltpu.SemaphoreType.DMA,
    ],
)
def _score_kernel(pois_hbm, urows_hbm, iemb_hbm,
                  pred_hbm, part_hbm,
                  pidx, urows, prows, tacc, predv, accv,
                  sem_u0, sem_u1, sem_p0, sem_p1):
    wid = lax.axis_index("s") * NC + lax.axis_index("c")
    ubase = wid * U_PER_W

    pltpu.sync_copy(pois_hbm.at[pl.ds(wid * PROWS_PW, PROWS_PW)], pidx)

    sems_u = (sem_u0, sem_u1)
    sems_p = (sem_p0, sem_p1)

    def copies(ch, b):
        cp_u = pltpu.make_async_copy(
            urows_hbm.at[pl.ds(ubase + ch * CU, CU)], urows.at[b], sems_u[b])
        cps_p = [
            pltpu.make_async_copy(
                iemb_hbm.at[pidx.at[ch * GSUB + k]],
                prows.at[b].at[pl.ds(k * GLEN, GLEN)], sems_p[b])
            for k in range(GSUB)
        ]
        return [cp_u] + cps_p

    def fire(ch, b):
        for cp in copies(ch, b):
            cp.start()

    def drain(ch, b):
        for cp in copies(ch, b):
            cp.wait()

    iota16 = lax.iota(jnp.int32, 16) * 16
    zero16 = jnp.zeros((16,), jnp.float32)

    def compute(ch, b, acc_i, acc_u):
        ur = urows.at[b]
        pr = prows.at[b]

        def user_body(c, carry):
            accs, acc_u = carry
            uv0 = ur[c, pl.ds(0, 16)]
            uv1 = ur[c, pl.ds(16, 16)]
            uv2 = ur[c, pl.ds(32, 16)]
            uv3 = ur[c, pl.ds(48, 16)]
            acc_u = acc_u + ((uv0 * uv0 + uv1 * uv1) + (uv2 * uv2 + uv3 * uv3))

            @plsc.parallel_loop(0, L, unroll=25, carry=accs)
            def accs(l, accs):
                a0, a1, a2, a3 = accs
                r = c * L + l
                pv0 = pr[r, pl.ds(0, 16)]
                pv1 = pr[r, pl.ds(16, 16)]
                pv2 = pr[r, pl.ds(32, 16)]
                pv3 = pr[r, pl.ds(48, 16)]
                tacc[pl.ds(r * 16, 16)] = (
                    (uv0 * pv0 + uv1 * pv1) + (uv2 * pv2 + uv3 * pv3))
                a0 = a0 + ((pv0 * pv0 + pv1 * pv1) + (pv2 * pv2 + pv3 * pv3))
                return (a1, a2, a3, a0)

            return accs, acc_u

        (accs, acc_u) = lax.fori_loop(
            0, CU, user_body, ((acc_i, zero16, zero16, zero16), acc_u))
        acc_i = (accs[0] + accs[1]) + (accs[2] + accs[3])

        @plsc.parallel_loop(0, GROUPS, unroll=25)
        def _(g):
            base = g * 256 + iota16
            gs = [plsc.load_gather(tacc, [base + k]) for k in range(16)]
            while len(gs) > 1:
                gs = [a + b for a, b in zip(gs[::2], gs[1::2])]
            predv[pl.ds(pl.multiple_of(g * 16, 16), 16)] = gs[0]

        pltpu.sync_copy(
            predv.at[pl.ds(0, ROWS)],
            pred_hbm.at[pl.ds((ubase + ch * CU) * L, ROWS)])
        return acc_i, acc_u

    fire(0, 0)

    def two_chunks(ch2, carry):
        acc_i, acc_u = carry
        for b in (0, 1):
            ch = ch2 * 2 + b

            @pl.when(ch + 1 < CHUNKS)
            def _():
                fire(ch + 1, 1 - b)

            drain(ch, b)
            acc_i, acc_u = compute(ch, b, acc_i, acc_u)
        return acc_i, acc_u

    zero = jnp.zeros((16,), jnp.float32)
    acc_i, acc_u = lax.fori_loop(0, CHUNKS // 2, two_chunks, (zero, zero))
    accv[...] = acc_i + jnp.float32(L) * acc_u
    pltpu.sync_copy(accv, part_hbm.at[wid])


def kernel(users, POIs, u_embeds, i_embeds):
    u_rows = jnp.take(u_embeds, users.reshape(-1), axis=0)
    pred_flat, partials = _score_kernel(
        POIs.reshape(B * L // GLEN, GLEN),
        u_rows,
        i_embeds,
    )
    pred = lax.optimization_barrier(pred_flat).reshape(B, L)
    l2_loss = jnp.sum(partials) / B
    reg_loss = LAM * l2_loss
    l2_loss_scaled = LAM * l2_loss
    return (pred, reg_loss, l2_loss, l2_loss_scaled)


# speedup vs baseline: 1.2432x; 1.2432x over previous
import functools

import jax
import jax.numpy as jnp
from jax import lax
from jax.experimental import pallas as pl
from jax.experimental.pallas import tpu as pltpu
from jax.experimental.pallas import tpu_sc as plsc

B = 16384
L = 50
D = 64
DP = D
LAM = 0.001

NC = 2
NS = 16
NW = NC * NS
U_PER_W = B // NW
CU = 8
ROWS = CU * L
CHUNKS = U_PER_W // CU
GSUB = 4
GLEN = ROWS // GSUB
PROWS_PW = U_PER_W * L // GLEN
GROUPS = (ROWS + 15) // 16
RPAD = GROUPS * 16

_mesh = plsc.VectorSubcoreMesh(core_axis_name="c", subcore_axis_name="s")


@functools.partial(
    pl.kernel,
    mesh=_mesh,
    compiler_params=pltpu.CompilerParams(
        needs_layout_passes=False, use_tc_tiling_on_sc=False),
    out_type=[
        jax.ShapeDtypeStruct((B * L,), jnp.float32),
        jax.ShapeDtypeStruct((NW, 16), jnp.float32),
    ],
    scratch_types=[
        pltpu.VMEM((PROWS_PW, GLEN), jnp.int32),
        pltpu.VMEM((2, CU, DP), jnp.float32),
        pltpu.VMEM((2, ROWS, DP), jnp.float32),
        pltpu.VMEM((RPAD * 16,), jnp.float32),
        pltpu.VMEM((RPAD,), jnp.float32),
        pltpu.VMEM((16,), jnp.float32),
        pltpu.SemaphoreType.DMA,
        pltpu.SemaphoreType.DMA,
        pltpu.SemaphoreType.DMA,
        pltpu.SemaphoreType.DMA,
    ],
)
def _score_kernel(pois_hbm, urows_hbm, iemb_hbm,
                  pred_hbm, part_hbm,
                  pidx, urows, prows, tacc, predv, accv,
                  sem_u0, sem_u1, sem_p0, sem_p1):
    wid = lax.axis_index("s") * NC + lax.axis_index("c")
    ubase = wid * U_PER_W

    pltpu.sync_copy(pois_hbm.at[pl.ds(wid * PROWS_PW, PROWS_PW)], pidx)

    sems_u = (sem_u0, sem_u1)
    sems_p = (sem_p0, sem_p1)

    def copies(ch, b):
        cp_u = pltpu.make_async_copy(
            urows_hbm.at[pl.ds(ubase + ch * CU, CU)], urows.at[b], sems_u[b])
        cps_p = [
            pltpu.make_async_copy(
                iemb_hbm.at[pidx.at[ch * GSUB + k]],
                prows.at[b].at[pl.ds(k * GLEN, GLEN)], sems_p[b])
            for k in range(GSUB)
        ]
        return [cp_u] + cps_p

    def fire(ch, b):
        for cp in copies(ch, b):
            cp.start()

    def drain(ch, b):
        for cp in copies(ch, b):
            cp.wait()

    iota16 = lax.iota(jnp.int32, 16) * 16
    zero16 = jnp.zeros((16,), jnp.float32)

    def compute(ch, b, acc_i, acc_u):
        ur = urows.at[b]
        pr = prows.at[b]

        def user_body(c, carry):
            accs, acc_u = carry
            uv0 = ur[c, pl.ds(0, 16)]
            uv1 = ur[c, pl.ds(16, 16)]
            uv2 = ur[c, pl.ds(32, 16)]
            uv3 = ur[c, pl.ds(48, 16)]
            acc_u = acc_u + ((uv0 * uv0 + uv1 * uv1) + (uv2 * uv2 + uv3 * uv3))

            @plsc.parallel_loop(0, L, unroll=10, carry=accs)
            def accs(l, accs):
                a0, a1, a2, a3 = accs
                r = c * L + l
                pv0 = pr[r, pl.ds(0, 16)]
                pv1 = pr[r, pl.ds(16, 16)]
                pv2 = pr[r, pl.ds(32, 16)]
                pv3 = pr[r, pl.ds(48, 16)]
                tacc[pl.ds(r * 16, 16)] = (
                    (uv0 * pv0 + uv1 * pv1) + (uv2 * pv2 + uv3 * pv3))
                a0 = a0 + ((pv0 * pv0 + pv1 * pv1) + (pv2 * pv2 + pv3 * pv3))
                return (a1, a2, a3, a0)

            return accs, acc_u

        (accs, acc_u) = lax.fori_loop(
            0, CU, user_body, ((acc_i, zero16, zero16, zero16), acc_u))
        acc_i = (accs[0] + accs[1]) + (accs[2] + accs[3])

        @plsc.parallel_loop(0, GROUPS, unroll=5)
        def _(g):
            base = g * 256 + iota16
            gs = [plsc.load_gather(tacc, [base + k]) for k in range(16)]
            while len(gs) > 1:
                gs = [a + b for a, b in zip(gs[::2], gs[1::2])]
            predv[pl.ds(pl.multiple_of(g * 16, 16), 16)] = gs[0]

        pltpu.sync_copy(
            predv.at[pl.ds(0, ROWS)],
            pred_hbm.at[pl.ds((ubase + ch * CU) * L, ROWS)])
        return acc_i, acc_u

    fire(0, 0)

    def two_chunks(ch2, carry):
        acc_i, acc_u = carry
        for b in (0, 1):
            ch = ch2 * 2 + b

            @pl.when(ch + 1 < CHUNKS)
            def _():
                fire(ch + 1, 1 - b)

            drain(ch, b)
            acc_i, acc_u = compute(ch, b, acc_i, acc_u)
        return acc_i, acc_u

    zero = jnp.zeros((16,), jnp.float32)
    acc_i, acc_u = lax.fori_loop(0, CHUNKS // 2, two_chunks, (zero, zero))
    accv[...] = acc_i + jnp.float32(L) * acc_u
    pltpu.sync_copy(accv, part_hbm.at[wid])


def kernel(users, POIs, u_embeds, i_embeds):
    u_rows = jnp.take(u_embeds, users.reshape(-1), axis=0)
    pred_flat, partials = _score_kernel(
        POIs.reshape(B * L // GLEN, GLEN),
        u_rows,
        i_embeds,
    )
    pred = lax.optimization_barrier(pred_flat).reshape(B, L)
    l2_loss = jnp.sum(partials) / B
    reg_loss = LAM * l2_loss
    l2_loss_scaled = LAM * l2_loss
    return (pred, reg_loss, l2_loss, l2_loss_scaled)


# speedup vs baseline: 1.2980x; 1.0441x over previous
import functools

import jax
import jax.numpy as jnp
from jax import lax
from jax.experimental import pallas as pl
from jax.experimental.pallas import tpu as pltpu
from jax.experimental.pallas import tpu_sc as plsc

B = 16384
L = 50
D = 64
DP = D
LAM = 0.001

NC = 2
NS = 16
NW = NC * NS
U_PER_W = B // NW
CU = 8
ROWS = CU * L
CHUNKS = U_PER_W // CU
GSUB = 4
GLEN = ROWS // GSUB
PROWS_PW = U_PER_W * L // GLEN
GROUPS = (ROWS + 15) // 16
RPAD = GROUPS * 16

_mesh = plsc.VectorSubcoreMesh(core_axis_name="c", subcore_axis_name="s")


@functools.partial(
    pl.kernel,
    mesh=_mesh,
    compiler_params=pltpu.CompilerParams(
        needs_layout_passes=False, use_tc_tiling_on_sc=False),
    out_type=[
        jax.ShapeDtypeStruct((B * L,), jnp.float32),
        jax.ShapeDtypeStruct((NW, 16), jnp.float32),
    ],
    scratch_types=[
        pltpu.VMEM((PROWS_PW, GLEN), jnp.int32),
        pltpu.VMEM((2, CU, DP), jnp.float32),
        pltpu.VMEM((2, ROWS, DP), jnp.float32),
        pltpu.VMEM((RPAD * 16,), jnp.float32),
        pltpu.VMEM((RPAD,), jnp.float32),
        pltpu.VMEM((16,), jnp.float32),
        pltpu.SemaphoreType.DMA,
        pltpu.SemaphoreType.DMA,
        pltpu.SemaphoreType.DMA,
        pltpu.SemaphoreType.DMA,
    ],
)
def _score_kernel(pois_hbm, urows_hbm, iemb_hbm,
                  pred_hbm, part_hbm,
                  pidx, urows, prows, tacc, predv, accv,
                  sem_u0, sem_u1, sem_p0, sem_p1):
    wid = lax.axis_index("s") * NC + lax.axis_index("c")
    ubase = wid * U_PER_W

    pltpu.sync_copy(pois_hbm.at[pl.ds(wid * PROWS_PW, PROWS_PW)], pidx)

    sems_u = (sem_u0, sem_u1)
    sems_p = (sem_p0, sem_p1)

    def copies(ch, b):
        cp_u = pltpu.make_async_copy(
            urows_hbm.at[pl.ds(ubase + ch * CU, CU)], urows.at[b], sems_u[b])
        cps_p = [
            pltpu.make_async_copy(
                iemb_hbm.at[pidx.at[ch * GSUB + k]],
                prows.at[b].at[pl.ds(k * GLEN, GLEN)], sems_p[b])
            for k in range(GSUB)
        ]
        return [cp_u] + cps_p

    def fire(ch, b):
        for cp in copies(ch, b):
            cp.start()

    def drain(ch, b):
        for cp in copies(ch, b):
            cp.wait()

    iota16 = lax.iota(jnp.int32, 16) * 16
    zero16 = jnp.zeros((16,), jnp.float32)

    def compute(ch, b, acc_i, acc_u):
        ur = urows.at[b]
        pr = prows.at[b]

        def user_body(c, carry):
            accs, acc_u = carry
            uv0 = ur[c, pl.ds(0, 16)]
            uv1 = ur[c, pl.ds(16, 16)]
            uv2 = ur[c, pl.ds(32, 16)]
            uv3 = ur[c, pl.ds(48, 16)]
            acc_u = acc_u + ((uv0 * uv0 + uv1 * uv1) + (uv2 * uv2 + uv3 * uv3))

            @plsc.parallel_loop(0, L, unroll=10, carry=accs)
            def accs(l, accs):
                a0, a1, a2, a3 = accs
                r = c * L + l
                pv0 = pr[r, pl.ds(0, 16)]
                pv1 = pr[r, pl.ds(16, 16)]
                pv2 = pr[r, pl.ds(32, 16)]
                pv3 = pr[r, pl.ds(48, 16)]
                tacc[pl.ds(r * 16, 16)] = (
                    (uv0 * pv0 + uv1 * pv1) + (uv2 * pv2 + uv3 * pv3))
                a0 = a0 + ((pv0 * pv0 + pv1 * pv1) + (pv2 * pv2 + pv3 * pv3))
                return (a1, a2, a3, a0)

            return accs, acc_u

        (accs, acc_u) = lax.fori_loop(
            0, CU, user_body, ((acc_i, zero16, zero16, zero16), acc_u))
        acc_i = (accs[0] + accs[1]) + (accs[2] + accs[3])

        @plsc.parallel_loop(0, GROUPS, unroll=5)
        def _(g):
            base = g * 256 + iota16
            gs = [plsc.load_gather(tacc, [base + k]) for k in range(16)]
            while len(gs) > 1:
                gs = [a + b for a, b in zip(gs[::2], gs[1::2])]
            predv[pl.ds(pl.multiple_of(g * 16, 16), 16)] = gs[0]

        pltpu.sync_copy(
            predv.at[pl.ds(0, ROWS)],
            pred_hbm.at[pl.ds((ubase + ch * CU) * L, ROWS)])
        return acc_i, acc_u

    fire(0, 0)

    def two_chunks(ch2, carry):
        acc_i, acc_u = carry
        for b in (0, 1):
            ch = ch2 * 2 + b

            @pl.when(ch + 1 < CHUNKS)
            def _():
                fire(ch + 1, 1 - b)

            drain(ch, b)
            acc_i, acc_u = compute(ch, b, acc_i, acc_u)
        return acc_i, acc_u

    zero = jnp.zeros((16,), jnp.float32)
    acc_i, acc_u = lax.fori_loop(0, CHUNKS // 2, two_chunks, (zero, zero))
    accv[...] = acc_i + jnp.float32(L) * acc_u
    pltpu.sync_copy(accv, part_hbm.at[wid])


def kernel(users, POIs, u_embeds, i_embeds):
    u_rows = u_embeds.at[users.reshape(-1)].get(mode='promise_in_bounds')
    pred_flat, partials = _score_kernel(
        POIs.reshape(B * L // GLEN, GLEN),
        u_rows,
        i_embeds,
    )
    pred = lax.optimization_barrier(pred_flat).reshape(B, L)
    l2_loss = jnp.sum(partials) / B
    reg_loss = LAM * l2_loss
    l2_loss_scaled = LAM * l2_loss
    return (pred, reg_loss, l2_loss, l2_loss_scaled)


# speedup vs baseline: 1.6083x; 1.2390x over previous
import functools

import jax
import jax.numpy as jnp
from jax import lax
from jax.experimental import pallas as pl
from jax.experimental.pallas import tpu as pltpu
from jax.experimental.pallas import tpu_sc as plsc

B = 16384
L = 50
D = 64
DP = D
LAM = 0.001

NC = 2
NS = 16
NW = NC * NS
U_PER_W = B // NW
CU = 8
ROWS = CU * L
CHUNKS = U_PER_W // CU
GSUB = 4
GLEN = ROWS // GSUB
PROWS_PW = U_PER_W * L // GLEN
GROUPS = (ROWS + 15) // 16
RPAD = GROUPS * 16

_mesh = plsc.VectorSubcoreMesh(core_axis_name="c", subcore_axis_name="s")


@functools.partial(
    pl.kernel,
    mesh=_mesh,
    compiler_params=pltpu.CompilerParams(
        needs_layout_passes=False, use_tc_tiling_on_sc=False),
    out_type=[
        jax.ShapeDtypeStruct((B * L,), jnp.float32),
        jax.ShapeDtypeStruct((NW, 16), jnp.float32),
    ],
    scratch_types=[
        pltpu.VMEM((PROWS_PW, GLEN), jnp.int32),
        pltpu.VMEM((2, CU, DP), jnp.float32),
        pltpu.VMEM((2, ROWS, DP), jnp.float32),
        pltpu.VMEM((RPAD * 16,), jnp.float32),
        pltpu.VMEM((RPAD,), jnp.float32),
        pltpu.VMEM((16,), jnp.float32),
        pltpu.SemaphoreType.DMA,
        pltpu.SemaphoreType.DMA,
        pltpu.SemaphoreType.DMA,
        pltpu.SemaphoreType.DMA,
    ],
)
def _score_kernel(pois_hbm, urows_hbm, iemb_hbm,
                  pred_hbm, part_hbm,
                  pidx, urows, prows, tacc, predv, accv,
                  sem_u0, sem_u1, sem_p0, sem_p1):
    wid = lax.axis_index("s") * NC + lax.axis_index("c")
    ubase = wid * U_PER_W

    pltpu.sync_copy(pois_hbm.at[pl.ds(wid * PROWS_PW, PROWS_PW)], pidx)

    sems_u = (sem_u0, sem_u1)
    sems_p = (sem_p0, sem_p1)

    def copies(ch, b):
        cp_u = pltpu.make_async_copy(
            urows_hbm.at[pl.ds(ubase + ch * CU, CU)], urows.at[b], sems_u[b])
        cps_p = [
            pltpu.make_async_copy(
                iemb_hbm.at[pidx.at[ch * GSUB + k]],
                prows.at[b].at[pl.ds(k * GLEN, GLEN)], sems_p[b])
            for k in range(GSUB)
        ]
        return [cp_u] + cps_p

    def fire(ch, b):
        for cp in copies(ch, b):
            cp.start()

    def drain(ch, b):
        for cp in copies(ch, b):
            cp.wait()

    iota16 = lax.iota(jnp.int32, 16) * 16
    zero16 = jnp.zeros((16,), jnp.float32)

    def compute(ch, b, acc_i, acc_u):
        ur = urows.at[b]
        pr = prows.at[b]

        def user_body(c, carry):
            accs, acc_u = carry
            uv0 = ur[c, pl.ds(0, 16)]
            uv1 = ur[c, pl.ds(16, 16)]
            uv2 = ur[c, pl.ds(32, 16)]
            uv3 = ur[c, pl.ds(48, 16)]
            acc_u = acc_u + ((uv0 * uv0 + uv1 * uv1) + (uv2 * uv2 + uv3 * uv3))

            @plsc.parallel_loop(0, L, unroll=10, carry=accs)
            def accs(l, accs):
                a0, a1, a2, a3 = accs
                r = c * L + l
                pv0 = pr[r, pl.ds(0, 16)]
                pv1 = pr[r, pl.ds(16, 16)]
                pv2 = pr[r, pl.ds(32, 16)]
                pv3 = pr[r, pl.ds(48, 16)]
                tacc[pl.ds(r * 16, 16)] = plsc.cumsum(
                    (uv0 * pv0 + uv1 * pv1) + (uv2 * pv2 + uv3 * pv3))
                a0 = a0 + ((pv0 * pv0 + pv1 * pv1) + (pv2 * pv2 + pv3 * pv3))
                return (a1, a2, a3, a0)

            return accs, acc_u

        (accs, acc_u) = lax.fori_loop(
            0, CU, user_body, ((acc_i, zero16, zero16, zero16), acc_u))
        acc_i = (accs[0] + accs[1]) + (accs[2] + accs[3])

        @plsc.parallel_loop(0, GROUPS, unroll=5)
        def _(g):
            s = plsc.load_gather(tacc, [g * 256 + iota16 + 15])
            predv[pl.ds(pl.multiple_of(g * 16, 16), 16)] = s

        pltpu.sync_copy(
            predv.at[pl.ds(0, ROWS)],
            pred_hbm.at[pl.ds((ubase + ch * CU) * L, ROWS)])
        return acc_i, acc_u

    fire(0, 0)

    def two_chunks(ch2, carry):
        acc_i, acc_u = carry
        for b in (0, 1):
            ch = ch2 * 2 + b

            @pl.when(ch + 1 < CHUNKS)
            def _():
                fire(ch + 1, 1 - b)

            drain(ch, b)
            acc_i, acc_u = compute(ch, b, acc_i, acc_u)
        return acc_i, acc_u

    zero = jnp.zeros((16,), jnp.float32)
    acc_i, acc_u = lax.fori_loop(0, CHUNKS // 2, two_chunks, (zero, zero))
    accv[...] = acc_i + jnp.float32(L) * acc_u
    pltpu.sync_copy(accv, part_hbm.at[wid])


def kernel(users, POIs, u_embeds, i_embeds):
    u_rows = u_embeds.at[users.reshape(-1)].get(mode='promise_in_bounds')
    pred_flat, partials = _score_kernel(
        POIs.reshape(B * L // GLEN, GLEN),
        u_rows,
        i_embeds,
    )
    pred = lax.optimization_barrier(pred_flat).reshape(B, L)
    l2_loss = jnp.sum(partials) / B
    reg_loss = LAM * l2_loss
    l2_loss_scaled = LAM * l2_loss
    return (pred, reg_loss, l2_loss, l2_loss_scaled)
